# Initial kernel scaffold; baseline (speedup 1.0000x reference)
import jax
import jax.numpy as jnp
from jax.experimental import pallas as pl


def kernel(features, edge_metapath_indices_0, edge_metapath_indices_1, target_idx_0, target_idx_1, r_vec, attn_a, fc1_W, fc1_b, fc2_w, fc_W, fc_b):
    raise NotImplementedError("write your pallas kernel here")


# speedup vs baseline: 2.6947x; 2.6947x over previous
import jax
import jax.numpy as jnp
from jax.experimental import pallas as pl

_N_NODES = 10000
_D = 128
_E = 160000
_L = 3
_B = 8192
_H = 8
_OUT_DIM = 64
_ATTN_DIM = 128
_N_REL = 2
_ETYPES = [[0, 1], [1, 0]]

_EBLK = 2000
_BBLK = 1024


def _encode_body(e0r, e0i, e1r, e1i, e2r, e2i, rot, a_rT, a_iT,
                 hr_out, hi_out, e_out):
    c0 = rot[0:1, :]
    s0 = rot[1:2, :]
    c1 = rot[2:3, :]
    s1 = rot[3:4, :]
    x0r, x0i = e0r[...], e0i[...]
    x1r, x1i = e1r[...], e1i[...]
    x2r, x2i = e2r[...], e2i[...]
    hr = (x0r * c0 - x0i * s0 + x1r * c1 - x1i * s1 + x2r) * (1.0 / 3.0)
    hi = (x0r * s0 + x0i * c0 + x1r * s1 + x1i * c1 + x2i) * (1.0 / 3.0)
    hr_out[...] = hr
    hi_out[...] = hi
    e = (jnp.dot(hr, a_rT[...], preferred_element_type=jnp.float32)
         + jnp.dot(hi, a_iT[...], preferred_element_type=jnp.float32))
    e_out[...] = jnp.where(e >= 0, e, 0.01 * e)


def _encode_call(ed_r, ed_i, rot, a_rT, a_iT):
    nblk = _E // _EBLK
    eb = pl.BlockSpec((_EBLK, _D // 2), lambda i: (i, 0))
    full = lambda shape: pl.BlockSpec(shape, lambda i: (0, 0))
    return pl.pallas_call(
        _encode_body,
        grid=(nblk,),
        in_specs=[eb, eb, eb, eb, eb, eb,
                  full((4, _D // 2)), full((_D // 2, _H)), full((_D // 2, _H))],
        out_specs=[eb, eb, pl.BlockSpec((_EBLK, _H), lambda i: (i, 0))],
        out_shape=[
            jax.ShapeDtypeStruct((_E, _D // 2), jnp.float32),
            jax.ShapeDtypeStruct((_E, _D // 2), jnp.float32),
            jax.ShapeDtypeStruct((_E, _H), jnp.float32),
        ],
    )(ed_r[0], ed_i[0], ed_r[1], ed_i[1], ed_r[2], ed_i[2], rot, a_rT, a_iT)


def _score_body(x_ref, w1_ref, b1_ref, w2_ref, h_out, part_out):
    h = x_ref[...]
    h = jnp.where(h > 0, h, jnp.exp(jnp.minimum(h, 0.0)) - 1.0)
    h_out[...] = h
    t = jnp.tanh(jnp.dot(h, w1_ref[...], preferred_element_type=jnp.float32)
                 + b1_ref[...])
    v = jnp.dot(t, w2_ref[...], preferred_element_type=jnp.float32)

    @pl.when(pl.program_id(0) == 0)
    def _():
        part_out[...] = jnp.zeros_like(part_out)

    part_out[...] += jnp.sum(v)


def _score_call(x, fc1_W, fc1_b, fc2_w):
    nblk = _B // _BBLK
    hd = _H * _D
    return pl.pallas_call(
        _score_body,
        grid=(nblk,),
        in_specs=[
            pl.BlockSpec((_BBLK, hd), lambda i: (i, 0)),
            pl.BlockSpec((hd, _ATTN_DIM), lambda i: (0, 0)),
            pl.BlockSpec((1, _ATTN_DIM), lambda i: (0, 0)),
            pl.BlockSpec((_ATTN_DIM, 1), lambda i: (0, 0)),
        ],
        out_specs=[
            pl.BlockSpec((_BBLK, hd), lambda i: (i, 0)),
            pl.BlockSpec((1, 128), lambda i: (0, 0)),
        ],
        out_shape=[
            jax.ShapeDtypeStruct((_B, hd), jnp.float32),
            jax.ShapeDtypeStruct((1, 128), jnp.float32),
        ],
    )(x, fc1_W, fc1_b.reshape(1, _ATTN_DIM), fc2_w.reshape(_ATTN_DIM, 1))


def _head_body(h0_ref, h1_ref, beta_ref, w_ref, b_ref, logits_out, h_out):
    b0 = beta_ref[0, 0]
    b1 = beta_ref[0, 1]
    hc = h0_ref[...] * b0 + h1_ref[...] * b1
    h_out[...] = hc
    logits_out[...] = (jnp.dot(hc, w_ref[...], preferred_element_type=jnp.float32)
                       + b_ref[...])


def _head_call(h0, h1, beta, fc_W, fc_b):
    nblk = _B // _BBLK
    hd = _H * _D
    return pl.pallas_call(
        _head_body,
        grid=(nblk,),
        in_specs=[
            pl.BlockSpec((_BBLK, hd), lambda i: (i, 0)),
            pl.BlockSpec((_BBLK, hd), lambda i: (i, 0)),
            pl.BlockSpec((1, 2), lambda i: (0, 0)),
            pl.BlockSpec((hd, _OUT_DIM), lambda i: (0, 0)),
            pl.BlockSpec((1, _OUT_DIM), lambda i: (0, 0)),
        ],
        out_specs=[
            pl.BlockSpec((_BBLK, _OUT_DIM), lambda i: (i, 0)),
            pl.BlockSpec((_BBLK, hd), lambda i: (i, 0)),
        ],
        out_shape=[
            jax.ShapeDtypeStruct((_B, _OUT_DIM), jnp.float32),
            jax.ShapeDtypeStruct((_B, hd), jnp.float32),
        ],
    )(h0, h1, beta.reshape(1, 2), fc_W, fc_b.reshape(1, _OUT_DIM))


def kernel(features, edge_metapath_indices_0, edge_metapath_indices_1,
           target_idx_0, target_idx_1, r_vec, attn_a, fc1_W, fc1_b, fc2_w,
           fc_W, fc_b):
    f32 = jnp.float32
    norm = jnp.sqrt(jnp.sum(r_vec ** 2, axis=-1, keepdims=True)) + 1e-12
    r = r_vec / norm
    conj = r * jnp.array([1.0, -1.0], dtype=r.dtype)
    r_exp = jnp.stack([r, conj], axis=1).reshape(2 * _N_REL, _D // 2, 2)

    feat_r = features[:, 0::2]
    feat_i = features[:, 1::2]

    idx_list = [edge_metapath_indices_0, edge_metapath_indices_1]
    seg_list = [target_idx_0, target_idx_1]

    h_list = []
    for p in range(2):
        rv1 = r_exp[_ETYPES[p][1]]
        c1, s1 = rv1[:, 0], rv1[:, 1]
        rv0 = r_exp[_ETYPES[p][0]]
        c0 = c1 * rv0[:, 0] - s1 * rv0[:, 1]
        s0 = c1 * rv0[:, 1] + s1 * rv0[:, 0]
        rot = jnp.stack([c0, s0, c1, s1]).astype(f32)

        idx = idx_list[p]
        ed_r = jnp.transpose(feat_r[idx], (1, 0, 2))
        ed_i = jnp.transpose(feat_i[idx], (1, 0, 2))

        a = attn_a[p]
        a_rT = jnp.transpose(a[:, 0::2]).astype(f32)
        a_iT = jnp.transpose(a[:, 1::2]).astype(f32)

        hid_r, hid_i, e = _encode_call(ed_r, ed_i, rot, a_rT, a_iT)
        hidden = jnp.stack([hid_r, hid_i], axis=-1).reshape(_E, _D)

        seg = seg_list[p]
        m = jax.ops.segment_max(e, seg, num_segments=_B,
                                indices_are_sorted=True)
        ex = jnp.exp(e - m[seg])
        denom = jax.ops.segment_sum(ex, seg, num_segments=_B,
                                    indices_are_sorted=True)
        alpha = ex / denom[seg]
        out = jax.ops.segment_sum(alpha[:, :, None] * hidden[:, None, :],
                                  seg, num_segments=_B,
                                  indices_are_sorted=True)
        h_list.append(out.reshape(_B, _H * _D))

    h0, part0 = _score_call(h_list[0], fc1_W, fc1_b, fc2_w)
    h1, part1 = _score_call(h_list[1], fc1_W, fc1_b, fc2_w)
    scores = jnp.stack([part0[0, 0], part1[0, 0]]) / _B
    beta = jax.nn.softmax(scores)

    logits, h = _head_call(h0, h1, beta, fc_W, fc_b)
    return logits, h
